# baseline (speedup 1.0000x reference)
import math

import jax
import jax.numpy as jnp
from jax.experimental import pallas as pl
from jax.experimental.pallas import tpu as pltpu

PHI = (1 + math.sqrt(5)) / 2
ALPHA = 1 / PHI
BETA = 1 / PHI ** 2

BM = 256


def _hv_kernel(
    signs_ref, bp_hbm, bm_hbm, out_ref, bp_v, bm_v, diff_ref, const_ref, sems
):
    L, _ = bp_v.shape

    @pl.when(pl.program_id(0) == 0)
    def _prep():
        cp_p = pltpu.make_async_copy(bp_hbm, bp_v, sems.at[0])
        cp_m = pltpu.make_async_copy(bm_hbm, bm_v, sems.at[1])
        cp_p.start()
        cp_m.start()
        cp_p.wait()
        cp_m.wait()
        w = jnp.where(
            jax.lax.broadcasted_iota(jnp.int32, (L, 1), 0) == 0, ALPHA, ALPHA + BETA
        )
        diff_ref[...] = ((bp_v[...] - bm_v[...]) * w).astype(jnp.bfloat16)
        const_ref[...] = jnp.sum(bm_v[...] * w, axis=0, keepdims=True)

    m = (signs_ref[...] == 1).astype(jnp.bfloat16)
    hv = (
        jnp.dot(m, diff_ref[...], preferred_element_type=jnp.float32)
        + const_ref[...]
    )
    norm = jnp.sqrt(jnp.sum(hv * hv, axis=1, keepdims=True))
    out_ref[...] = jnp.where(norm > 0, hv / jnp.maximum(norm, 1e-12), hv)


def kernel(base_plus, base_minus, signs):
    B, L = signs.shape
    D = base_plus.shape[1]
    return pl.pallas_call(
        _hv_kernel,
        grid=(B // BM,),
        in_specs=[
            pl.BlockSpec((BM, L), lambda i: (i, 0)),
            pl.BlockSpec(memory_space=pl.ANY),
            pl.BlockSpec(memory_space=pl.ANY),
        ],
        out_specs=pl.BlockSpec((BM, D), lambda i: (i, 0)),
        out_shape=jax.ShapeDtypeStruct((B, D), jnp.float32),
        scratch_shapes=[
            pltpu.VMEM((L, D), jnp.float32),
            pltpu.VMEM((L, D), jnp.float32),
            pltpu.VMEM((L, D), jnp.bfloat16),
            pltpu.VMEM((1, D), jnp.float32),
            pltpu.SemaphoreType.DMA((2,)),
        ],
    )(signs, base_plus, base_minus)


# speedup vs baseline: 1.0504x; 1.0504x over previous
import math

import jax
import jax.numpy as jnp
from jax.experimental import pallas as pl
from jax.experimental.pallas import tpu as pltpu

PHI = (1 + math.sqrt(5)) / 2
ALPHA = 1 / PHI
BETA = 1 / PHI ** 2

BM = 1024


def _hv_kernel(
    signs_ref, bp_hbm, bm_hbm, out_ref, bp_v, bm_v, diff_ref, const_ref, sems
):
    L, _ = bp_v.shape

    @pl.when(pl.program_id(0) == 0)
    def _prep():
        cp_p = pltpu.make_async_copy(bp_hbm, bp_v, sems.at[0])
        cp_m = pltpu.make_async_copy(bm_hbm, bm_v, sems.at[1])
        cp_p.start()
        cp_m.start()
        cp_p.wait()
        cp_m.wait()
        w = jnp.where(
            jax.lax.broadcasted_iota(jnp.int32, (L, 1), 0) == 0, ALPHA, ALPHA + BETA
        )
        diff_ref[...] = ((bp_v[...] - bm_v[...]) * w).astype(jnp.bfloat16)
        const_ref[...] = jnp.sum(bm_v[...] * w, axis=0, keepdims=True)

    m = (signs_ref[...] == 1).astype(jnp.bfloat16)
    hv = (
        jnp.dot(m, diff_ref[...], preferred_element_type=jnp.float32)
        + const_ref[...]
    )
    norm = jnp.sqrt(jnp.sum(hv * hv, axis=1, keepdims=True))
    out_ref[...] = jnp.where(norm > 0, hv / jnp.maximum(norm, 1e-12), hv)


def kernel(base_plus, base_minus, signs):
    B, L = signs.shape
    D = base_plus.shape[1]
    return pl.pallas_call(
        _hv_kernel,
        grid=(B // BM,),
        in_specs=[
            pl.BlockSpec((BM, L), lambda i: (i, 0)),
            pl.BlockSpec(memory_space=pl.ANY),
            pl.BlockSpec(memory_space=pl.ANY),
        ],
        out_specs=pl.BlockSpec((BM, D), lambda i: (i, 0)),
        out_shape=jax.ShapeDtypeStruct((B, D), jnp.float32),
        scratch_shapes=[
            pltpu.VMEM((L, D), jnp.float32),
            pltpu.VMEM((L, D), jnp.float32),
            pltpu.VMEM((L, D), jnp.bfloat16),
            pltpu.VMEM((1, D), jnp.float32),
            pltpu.SemaphoreType.DMA((2,)),
        ],
    )(signs, base_plus, base_minus)


# speedup vs baseline: 1.0752x; 1.0236x over previous
import math

import jax
import jax.numpy as jnp
from jax.experimental import pallas as pl
from jax.experimental.pallas import tpu as pltpu

PHI = (1 + math.sqrt(5)) / 2
ALPHA = 1 / PHI
BETA = 1 / PHI ** 2

BM = 1024


def _hv_kernel(
    signs_ref, bp_hbm, bm_hbm, out_ref, bp_v, bm_v, diff_ref, const_ref, sems
):
    L, _ = bp_v.shape

    @pl.when(pl.program_id(0) == 0)
    def _prep():
        cp_p = pltpu.make_async_copy(bp_hbm, bp_v, sems.at[0])
        cp_m = pltpu.make_async_copy(bm_hbm, bm_v, sems.at[1])
        cp_p.start()
        cp_m.start()
        cp_p.wait()
        cp_m.wait()
        w = jnp.where(
            jax.lax.broadcasted_iota(jnp.int32, (L, 1), 0) == 0, ALPHA, ALPHA + BETA
        )
        diff_ref[...] = ((bp_v[...] - bm_v[...]) * w).astype(jnp.bfloat16)
        const_ref[...] = jnp.sum(bm_v[...] * w, axis=0, keepdims=True)

    m = signs_ref[...].astype(jnp.bfloat16)
    hv = (
        jnp.dot(m, diff_ref[...], preferred_element_type=jnp.float32)
        + const_ref[...]
    )
    ssq = jnp.sum(hv * hv, axis=1, keepdims=True)
    inv = jnp.where(ssq > 0, jax.lax.rsqrt(ssq), 1.0)
    out_ref[...] = hv * inv


def kernel(base_plus, base_minus, signs):
    B, L = signs.shape
    D = base_plus.shape[1]
    return pl.pallas_call(
        _hv_kernel,
        grid=(B // BM,),
        in_specs=[
            pl.BlockSpec((BM, L), lambda i: (i, 0)),
            pl.BlockSpec(memory_space=pl.ANY),
            pl.BlockSpec(memory_space=pl.ANY),
        ],
        out_specs=pl.BlockSpec((BM, D), lambda i: (i, 0)),
        out_shape=jax.ShapeDtypeStruct((B, D), jnp.float32),
        scratch_shapes=[
            pltpu.VMEM((L, D), jnp.float32),
            pltpu.VMEM((L, D), jnp.float32),
            pltpu.VMEM((L, D), jnp.bfloat16),
            pltpu.VMEM((1, D), jnp.float32),
            pltpu.SemaphoreType.DMA((2,)),
        ],
    )(signs, base_plus, base_minus)
